# baseline (speedup 1.0000x reference)
import functools

import jax
import jax.numpy as jnp
from jax import lax
from jax.experimental import pallas as pl
from jax.experimental.pallas import tpu as pltpu
from jax.experimental.pallas import tpu_sc as plsc

NUM_EMB = 1_000_000
DIM = 64
BATCH = 16384
HIST = 20
B_TOTAL = BATCH * HIST
NW = 32
B_PER_W = BATCH // NW
CHUNK = 128
NCHUNK = HIST * (B_PER_W // CHUNK)
BC_PER_W = B_PER_W // CHUNK
NBUF = 4


def _make_kernel():
    mesh = plsc.VectorSubcoreMesh(core_axis_name="c", subcore_axis_name="s")

    @functools.partial(
        pl.kernel,
        mesh=mesh,
        out_type=jax.ShapeDtypeStruct(
            (HIST * DIM // 8, BATCH // CHUNK, 8, CHUNK), jnp.float32
        ),
        scratch_types=[
            pltpu.VMEM((NCHUNK, CHUNK), jnp.int32),
            pltpu.VMEM((NCHUNK * CHUNK,), jnp.int32),
            pltpu.VMEM((NBUF, CHUNK, 2 * DIM), jnp.float32),
            pltpu.VMEM((NBUF, 8, 8, CHUNK), jnp.float32),
            pltpu.SemaphoreType.DMA((NBUF,)),
            pltpu.SemaphoreType.DMA((NBUF,)),
        ],
        compiler_params=pltpu.CompilerParams(
            use_tc_tiling_on_sc=False, needs_layout_passes=False
        ),
    )
    def gather_kernel(idxp_hbm, pb_hbm, table_hbm, out_hbm,
                      idxp_v, pb_v, rows_v, outT_v, gsem, ssem):
        wid = lax.axis_index("s") * 2 + lax.axis_index("c")
        pltpu.sync_copy(idxp_hbm.at[wid], idxp_v)
        pltpu.sync_copy(pb_hbm.at[wid], pb_v)

        ridx = [
            lax.iota(jnp.int32, 16) + kg * 16 for kg in range(8)
        ]

        for b in range(NBUF):
            pltpu.async_copy(
                table_hbm.at[idxp_v.at[b]], rows_v.at[b], gsem.at[b]
            )

        def body(c, carry):
            rb = lax.rem(c, NBUF)
            t = c // BC_PER_W
            bcg = wid * BC_PER_W + lax.rem(c, BC_PER_W)
            pltpu.make_async_copy(
                table_hbm.at[pl.ds(0, CHUNK)], rows_v.at[rb], gsem.at[rb]
            ).wait()
            @pl.when(c >= NBUF)
            def _():
                pltpu.make_async_copy(
                    outT_v.at[rb], out_hbm.at[pl.ds(0, 8), 0], ssem.at[rb]
                ).wait()
            rows = rows_v.at[rb]
            for kg in range(8):
                pb = pb_v[pl.ds(c * CHUNK + kg * 16, 16)]
                for d in range(DIM):
                    g = plsc.load_gather(rows, [ridx[kg], pb + d])
                    outT_v[rb, d // 8, d % 8, pl.ds(kg * 16, 16)] = g
            pltpu.async_copy(
                outT_v.at[rb], out_hbm.at[pl.ds(t * 8, 8), bcg], ssem.at[rb]
            )
            @pl.when(c + NBUF < NCHUNK)
            def _():
                pltpu.async_copy(
                    table_hbm.at[idxp_v.at[c + NBUF]], rows_v.at[rb],
                    gsem.at[rb],
                )
            return carry

        lax.fori_loop(0, NCHUNK, body, 0)
        for b in range(NBUF):
            pltpu.make_async_copy(
                outT_v.at[b], out_hbm.at[pl.ds(0, 8), 0], ssem.at[b]
            ).wait()

    return gather_kernel


_gather = _make_kernel()


def kernel(x, weight):
    xt = x.T.astype(jnp.int32)
    xw = xt.reshape(HIST, NW, BC_PER_W, CHUNK)
    xw = xw.transpose(1, 0, 2, 3).reshape(NW, NCHUNK, CHUNK)
    idxp = xw >> 1
    pb = (xw & 1).reshape(NW, NCHUNK * CHUNK) * DIM
    out = _gather(idxp, pb, weight.reshape(NUM_EMB // 2, 2 * DIM))
    v = out.reshape(HIST, 8, BATCH // CHUNK, 8, CHUNK)
    return v.transpose(2, 4, 0, 1, 3).reshape(BATCH, HIST, DIM)
